# baseline (speedup 1.0000x reference)
import jax
import jax.numpy as jnp
from jax import lax
from jax.experimental import pallas as pl
from jax.experimental.pallas import tpu as pltpu
from jax.experimental.pallas import tpu_sc as plsc

import functools

NP = 102400
L = 128
NW = 32
CK = 32
ZR = NP // 16


def _sc_agg_body(xq_h, src_h, dst_h, w_h, zeros_h, out,
                 xq0, xq1, a0, a1, a2, a3, a4,
                 s0, s1, d0, d1, w0, w1,
                 q00, q01, q10, q11,
                 c00, c01, c02, c03, c10, c11, c12, c13, ones_v,
                 gsem0, gsem1, ssem0, ssem1, esem0, esem1,
                 tr, nchunk):
    c = lax.axis_index("c")
    s = lax.axis_index("s")
    xs = [xq0, xq1]
    ac = [a0, a1, a2, a3, a4]
    srcb = [s0, s1]
    dstb = [d0, d1]
    wb = [w0, w1]
    qcols = [[q00, q01], [q10, q11]]
    cols = [[c00, c01, c02, c03], [c10, c11, c12, c13]]
    gsem = [gsem0, gsem1]
    ssem = [ssem0, ssem1]
    esem = [esem0, esem1]
    ce = CK * L

    for f in range(2):
        pltpu.sync_copy(xq_h.at[f, pl.ds(s * ZR, ZR)],
                        xs[f].at[pl.ds(s * ZR, ZR)])
    for f in range(5):
        pltpu.sync_copy(zeros_h, ac[f].at[pl.ds(s * ZR, ZR)])
    i16 = lax.broadcasted_iota(jnp.int32, (16,), 0)
    one16 = (i16 * 0 + 1).astype(jnp.float32)

    def fill(u, carry):
        ones_v[pl.ds(u * 16, 16)] = one16
        return carry

    lax.fori_loop(0, ce // 16, fill, 0)
    plsc.subcore_barrier()

    wid = s * 2 + c
    ebase0 = wid * tr * L

    def fire_stage(ci, b):
        base = ebase0 + ci * ce
        pltpu.async_copy(src_h.at[pl.ds(base, ce)], srcb[b], esem[b])
        pltpu.async_copy(dst_h.at[pl.ds(base, ce)], dstb[b], esem[b])
        pltpu.async_copy(w_h.at[pl.ds(base, ce)], wb[b], esem[b])

    def wait_stage(b):
        pltpu.make_async_copy(src_h.at[pl.ds(0, ce)], srcb[b], esem[b]).wait()
        pltpu.make_async_copy(dst_h.at[pl.ds(0, ce)], dstb[b], esem[b]).wait()
        pltpu.make_async_copy(w_h.at[pl.ds(0, ce)], wb[b], esem[b]).wait()

    def fire_gathers(b):
        for f in range(2):
            pltpu.async_copy(xs[f].at[srcb[b]], qcols[b][f], gsem[b])

    def wait_gathers(b):
        for f in range(2):
            pltpu.make_async_copy(xs[f].at[srcb[b]], qcols[b][f],
                                  gsem[b]).wait()

    def fire_scatters(b):
        for f in range(4):
            pltpu.async_copy(cols[b][f], ac[f].at[dstb[b]], ssem[b],
                             add=True)
        pltpu.async_copy(ones_v, ac[4].at[dstb[b]], ssem[b], add=True)

    def wait_scatters(b):
        for f in range(4):
            pltpu.make_async_copy(cols[b][f], ac[f].at[dstb[b]],
                                  ssem[b]).wait()
        pltpu.make_async_copy(ones_v, ac[4].at[dstb[b]], ssem[b]).wait()

    i16v = lax.broadcasted_iota(jnp.int32, (16,), 0)
    himask = i16v * 0 + (-65536)
    sh16 = i16v * 0 + 16

    def multiply(b):
        def mul_body(j, carry2):
            for u in range(L // 16):
                o = j * L + u * 16
                wv = wb[b][pl.ds(o, 16)]
                for p in range(2):
                    q = qcols[b][p][pl.ds(o, 16)]
                    hi = plsc.bitcast(q & himask, jnp.float32)
                    lo = plsc.bitcast(q << sh16, jnp.float32)
                    cols[b][2 * p][pl.ds(o, 16)] = hi * wv
                    cols[b][2 * p + 1][pl.ds(o, 16)] = lo * wv
            return carry2

        lax.fori_loop(0, CK, mul_body, 0)

    def step(ci, b, first):
        b2 = 1 - b
        if not first:
            wait_scatters(b2)

        @pl.when(ci + 1 < nchunk)
        def _():
            fire_stage(ci + 1, b2)

        wait_gathers(b)
        multiply(b)
        fire_scatters(b)

        @pl.when(ci + 1 < nchunk)
        def _():
            wait_stage(b2)
            fire_gathers(b2)

    fire_stage(0, 0)
    wait_stage(0)
    fire_gathers(0)
    step(0, 0, True)

    def pair_body(k, carry):
        step(2 * k + 1, 1, False)
        step(2 * k + 2, 0, False)
        return carry

    lax.fori_loop(0, (nchunk - 1) // 2, pair_body, 0)
    wait_scatters(0)

    plsc.subcore_barrier()
    for f in range(5):
        pltpu.sync_copy(ac[f].at[pl.ds(s * ZR, ZR)],
                        out.at[c * 5 + f, pl.ds(s * ZR, ZR)])


def _make_sc_agg(tr):
    mesh = plsc.VectorSubcoreMesh(core_axis_name="c", subcore_axis_name="s",
                                  num_cores=2, num_subcores=16)
    body = functools.partial(_sc_agg_body, tr=tr, nchunk=tr // CK)
    return pl.kernel(
        body,
        out_type=jax.ShapeDtypeStruct((10, NP), jnp.float32),
        mesh=mesh,
        compiler_params=pltpu.CompilerParams(needs_layout_passes=False),
        scratch_types=(
            [pltpu.VMEM_SHARED((NP,), jnp.int32) for _ in range(2)]
            + [pltpu.VMEM_SHARED((NP,), jnp.float32) for _ in range(5)]
            + [pltpu.VMEM((CK * L,), jnp.int32) for _ in range(2)]
            + [pltpu.VMEM((CK * L,), jnp.int32) for _ in range(2)]
            + [pltpu.VMEM((CK * L,), jnp.float32) for _ in range(2)]
            + [pltpu.VMEM((CK * L,), jnp.int32) for _ in range(4)]
            + [pltpu.VMEM((CK * L,), jnp.float32) for _ in range(8)]
            + [pltpu.VMEM((CK * L,), jnp.float32)]
            + [pltpu.SemaphoreType.DMA for _ in range(6)]),
        name="gcn_edge_aggregate",
    )


def _dense_body(parts_ref, xt_ref, wg_ref,
                wri_ref, wrh_ref, br_ref, wzi_ref, wzh_ref, bz_ref,
                wni_ref, bni_ref, wnh_ref, bnh_ref,
                wi_ref, bi_ref, wg2_ref, bg2_ref, wo_ref, bo_ref,
                lw_ref, lb_ref, out_ref):
    seg = parts_ref[0] + parts_ref[1]
    cnt = jnp.clip(seg[4:5], 1.0, None)

    def mm(w_ref, v):
        return lax.dot_general(w_ref[...], v, (((1,), (0,)), ((), ())),
                               preferred_element_type=jnp.float32)

    agg = mm(wg_ref, seg[0:4] / cnt)
    xt = xt_ref[...]
    r = jax.nn.sigmoid(mm(wri_ref, agg) + mm(wrh_ref, xt) + br_ref[...])
    z = jax.nn.sigmoid(mm(wzi_ref, agg) + mm(wzh_ref, xt) + bz_ref[...])
    n = jnp.tanh(mm(wni_ref, agg) + bni_ref[...]
                 + r * (mm(wnh_ref, xt) + bnh_ref[...]))
    h = (1.0 - z) * n + z * xt
    ig = jax.nn.sigmoid(mm(wi_ref, h) + bi_ref[...])
    gg = jnp.tanh(mm(wg2_ref, h) + bg2_ref[...])
    og = jax.nn.sigmoid(mm(wo_ref, h) + bo_ref[...])
    hout = og * jnp.tanh(ig * gg)
    out_ref[...] = (lax.dot_general(lw_ref[...], jnp.maximum(hout, 0.0),
                                    (((1,), (0,)), ((), ())),
                                    preferred_element_type=jnp.float32)
                    + lb_ref[...])


def _dense_call(parts, xt, consts, bn):
    grid = NP // bn
    small = [pl.BlockSpec(c.shape, lambda i, nd=c.ndim: (0,) * nd)
             for c in consts]
    return pl.pallas_call(
        _dense_body,
        grid=(grid,),
        in_specs=[
            pl.BlockSpec((2, 5, bn), lambda i: (0, 0, i)),
            pl.BlockSpec((4, bn), lambda i: (0, i)),
        ] + small,
        out_specs=pl.BlockSpec((1, bn), lambda i: (0, i)),
        out_shape=jax.ShapeDtypeStruct((1, NP), jnp.float32),
    )(parts, xt, *consts)


def kernel(x, edge_index, edge_weight, ggc_weight, gru_w_ih, gru_w_hh,
           gru_b_ih, gru_b_hh, lstm_w_ih, lstm_w_hh, lstm_b_ih, lstm_b_hh,
           lin_w, lin_b):
    n, f = x.shape
    e = edge_weight.shape[0]

    xt = jnp.zeros((4, NP), jnp.float32).at[:, :n].set(x.T)

    nr = -(-e // L)
    tr = -(-(-(-nr // NW)) // CK) * CK
    if (tr // CK) % 2 == 0:
        tr += CK
    nr2 = NW * tr
    pad_e = nr2 * L - e
    src = jnp.concatenate([edge_index[0], jnp.zeros((pad_e,), jnp.int32)])
    dst = jnp.concatenate([edge_index[1],
                           jnp.full((pad_e,), NP - 1, jnp.int32)])
    w = jnp.concatenate([edge_weight, jnp.zeros((pad_e,), jnp.float32)])
    zeros = jnp.zeros((ZR,), jnp.float32)

    xb = lax.bitcast_convert_type(x.astype(jnp.bfloat16),
                                  jnp.uint16).astype(jnp.uint32)
    xq = jnp.zeros((2, NP), jnp.uint32)
    xq = xq.at[0, :n].set((xb[:, 0] << 16) | xb[:, 1])
    xq = xq.at[1, :n].set((xb[:, 2] << 16) | xb[:, 3])
    xq = lax.bitcast_convert_type(xq, jnp.int32)
    parts = _make_sc_agg(tr)(xq, src, dst, w, zeros)
    parts = parts.reshape(2, 5, NP)

    col = lambda v: v.reshape(-1, 1)
    consts = [
        ggc_weight.T,
        gru_w_ih[0:4], gru_w_hh[0:4], col(gru_b_ih[0:4] + gru_b_hh[0:4]),
        gru_w_ih[4:8], gru_w_hh[4:8], col(gru_b_ih[4:8] + gru_b_hh[4:8]),
        gru_w_ih[8:12], col(gru_b_ih[8:12]),
        gru_w_hh[8:12], col(gru_b_hh[8:12]),
        lstm_w_ih[0:32], col(lstm_b_ih[0:32] + lstm_b_hh[0:32]),
        lstm_w_ih[64:96], col(lstm_b_ih[64:96] + lstm_b_hh[64:96]),
        lstm_w_ih[96:128], col(lstm_b_ih[96:128] + lstm_b_hh[96:128]),
        lin_w, lin_b.reshape(1, 1),
    ]
    out_t = _dense_call(parts, xt, consts, bn=2048)
    return out_t.reshape(NP, 1)[:n]


# speedup vs baseline: 1.1535x; 1.1535x over previous
import jax
import jax.numpy as jnp
from jax import lax
from jax.experimental import pallas as pl
from jax.experimental.pallas import tpu as pltpu
from jax.experimental.pallas import tpu_sc as plsc

import functools

NP = 102400
L = 128
NW = 32
CK = 32
ZR = NP // 16


def _sc_agg_body(xq_h, src_h, dst_h, w_h, zeros_h, out,
                 xq0, xq1, a0, a1, a2, a3, a4,
                 s0, s1, d0, d1, w0, w1,
                 q00, q01, q10, q11,
                 c00, c01, c02, c03, c10, c11, c12, c13, ones_v,
                 gsem0, gsem1, ssem0, ssem1, esem0, esem1,
                 tr, nchunk):
    c = lax.axis_index("c")
    s = lax.axis_index("s")
    xs = [xq0, xq1]
    ac = [a0, a1, a2, a3, a4]
    srcb = [s0, s1]
    dstb = [d0, d1]
    wb = [w0, w1]
    qcols = [[q00, q01], [q10, q11]]
    cols = [[c00, c01, c02, c03], [c10, c11, c12, c13]]
    gsem = [gsem0, gsem1]
    ssem = [ssem0, ssem1]
    esem = [esem0, esem1]
    ce = CK * L

    for f in range(2):
        pltpu.sync_copy(xq_h.at[f, pl.ds(s * ZR, ZR)],
                        xs[f].at[pl.ds(s * ZR, ZR)])
    for f in range(5):
        pltpu.sync_copy(zeros_h, ac[f].at[pl.ds(s * ZR, ZR)])
    i16 = lax.broadcasted_iota(jnp.int32, (16,), 0)
    one16 = (i16 * 0 + 1).astype(jnp.float32)

    def fill(u, carry):
        ones_v[pl.ds(u * 16, 16)] = one16
        return carry

    lax.fori_loop(0, ce // 16, fill, 0)
    plsc.subcore_barrier()

    wid = s * 2 + c
    ebase0 = wid * tr * L

    def fire_stage(ci, b):
        base = ebase0 + ci * ce
        pltpu.async_copy(src_h.at[pl.ds(base, ce)], srcb[b], esem[b])
        pltpu.async_copy(dst_h.at[pl.ds(base, ce)], dstb[b], esem[b])
        pltpu.async_copy(w_h.at[pl.ds(base, ce)], wb[b], esem[b])

    def wait_stage(b):
        pltpu.make_async_copy(src_h.at[pl.ds(0, ce)], srcb[b], esem[b]).wait()
        pltpu.make_async_copy(dst_h.at[pl.ds(0, ce)], dstb[b], esem[b]).wait()
        pltpu.make_async_copy(w_h.at[pl.ds(0, ce)], wb[b], esem[b]).wait()

    def fire_gathers(b):
        for f in range(2):
            pltpu.async_copy(xs[f].at[srcb[b]], qcols[b][f], gsem[b])

    def wait_gathers(b):
        for f in range(2):
            pltpu.make_async_copy(xs[f].at[srcb[b]], qcols[b][f],
                                  gsem[b]).wait()

    def fire_scatters(b):
        for f in range(4):
            pltpu.async_copy(cols[b][f], ac[f].at[dstb[b]], ssem[b],
                             add=True)
        pltpu.async_copy(ones_v, ac[4].at[dstb[b]], ssem[b], add=True)

    def wait_scatters(b):
        for f in range(4):
            pltpu.make_async_copy(cols[b][f], ac[f].at[dstb[b]],
                                  ssem[b]).wait()
        pltpu.make_async_copy(ones_v, ac[4].at[dstb[b]], ssem[b]).wait()

    i16v = lax.broadcasted_iota(jnp.int32, (16,), 0)
    himask = i16v * 0 + (-65536)
    sh16 = i16v * 0 + 16

    def multiply(b):
        def mul_body(j, carry2):
            for u in range(L // 16):
                o = j * L + u * 16
                wv = wb[b][pl.ds(o, 16)]
                for p in range(2):
                    q = qcols[b][p][pl.ds(o, 16)]
                    hi = plsc.bitcast(q & himask, jnp.float32)
                    lo = plsc.bitcast(q << sh16, jnp.float32)
                    cols[b][2 * p][pl.ds(o, 16)] = hi * wv
                    cols[b][2 * p + 1][pl.ds(o, 16)] = lo * wv
            return carry2

        lax.fori_loop(0, CK, mul_body, 0)

    def step(ci, b, first):
        b2 = 1 - b
        if not first:
            wait_scatters(b2)

        @pl.when(ci + 1 < nchunk)
        def _():
            fire_stage(ci + 1, b2)

        wait_gathers(b)
        multiply(b)
        fire_scatters(b)

        @pl.when(ci + 1 < nchunk)
        def _():
            wait_stage(b2)
            fire_gathers(b2)

    fire_stage(0, 0)
    wait_stage(0)
    fire_gathers(0)
    step(0, 0, True)

    def pair_body(k, carry):
        step(2 * k + 1, 1, False)
        step(2 * k + 2, 0, False)
        return carry

    lax.fori_loop(0, (nchunk - 1) // 2, pair_body, 0)
    wait_scatters(0)

    plsc.subcore_barrier()
    for f in range(5):
        pltpu.sync_copy(ac[f].at[pl.ds(s * ZR, ZR)],
                        out.at[c * 5 + f, pl.ds(s * ZR, ZR)])


def _make_sc_agg(tr):
    mesh = plsc.VectorSubcoreMesh(core_axis_name="c", subcore_axis_name="s",
                                  num_cores=2, num_subcores=16)
    body = functools.partial(_sc_agg_body, tr=tr, nchunk=tr // CK)
    return pl.kernel(
        body,
        out_type=jax.ShapeDtypeStruct((10, NP), jnp.float32),
        mesh=mesh,
        compiler_params=pltpu.CompilerParams(needs_layout_passes=False),
        scratch_types=(
            [pltpu.VMEM_SHARED((NP,), jnp.int32) for _ in range(2)]
            + [pltpu.VMEM_SHARED((NP,), jnp.float32) for _ in range(5)]
            + [pltpu.VMEM((CK * L,), jnp.int32) for _ in range(2)]
            + [pltpu.VMEM((CK * L,), jnp.int32) for _ in range(2)]
            + [pltpu.VMEM((CK * L,), jnp.float32) for _ in range(2)]
            + [pltpu.VMEM((CK * L,), jnp.int32) for _ in range(4)]
            + [pltpu.VMEM((CK * L,), jnp.float32) for _ in range(8)]
            + [pltpu.VMEM((CK * L,), jnp.float32)]
            + [pltpu.SemaphoreType.DMA for _ in range(6)]),
        name="gcn_edge_aggregate",
    )


def _dense_body(parts_ref, xt_ref, wg_ref,
                wri_ref, wrh_ref, br_ref, wzi_ref, wzh_ref, bz_ref,
                wni_ref, bni_ref, wnh_ref, bnh_ref,
                wi_ref, bi_ref, wg2_ref, bg2_ref, wo_ref, bo_ref,
                lw_ref, lb_ref, out_ref):
    seg = parts_ref[0] + parts_ref[1]
    cnt = jnp.clip(seg[4:5], 1.0, None)

    def mm(w_ref, v):
        return lax.dot_general(w_ref[...], v, (((1,), (0,)), ((), ())),
                               preferred_element_type=jnp.float32)

    agg = mm(wg_ref, seg[0:4] / cnt)
    xt = xt_ref[...]
    r = jax.nn.sigmoid(mm(wri_ref, agg) + mm(wrh_ref, xt) + br_ref[...])
    z = jax.nn.sigmoid(mm(wzi_ref, agg) + mm(wzh_ref, xt) + bz_ref[...])
    n = jnp.tanh(mm(wni_ref, agg) + bni_ref[...]
                 + r * (mm(wnh_ref, xt) + bnh_ref[...]))
    h = (1.0 - z) * n + z * xt
    ig = jax.nn.sigmoid(mm(wi_ref, h) + bi_ref[...])
    gg = jnp.tanh(mm(wg2_ref, h) + bg2_ref[...])
    og = jax.nn.sigmoid(mm(wo_ref, h) + bo_ref[...])
    hout = og * jnp.tanh(ig * gg)
    out_ref[...] = (lax.dot_general(lw_ref[...], jnp.maximum(hout, 0.0),
                                    (((1,), (0,)), ((), ())),
                                    preferred_element_type=jnp.float32)
                    + lb_ref[...])


def _dense_call(parts, xt, consts, bn):
    grid = NP // bn
    small = [pl.BlockSpec(c.shape, lambda i, nd=c.ndim: (0,) * nd)
             for c in consts]
    return pl.pallas_call(
        _dense_body,
        grid=(grid,),
        in_specs=[
            pl.BlockSpec((2, 5, bn), lambda i: (0, 0, i)),
            pl.BlockSpec((4, bn), lambda i: (0, i)),
        ] + small,
        out_specs=pl.BlockSpec((1, bn), lambda i: (0, i)),
        out_shape=jax.ShapeDtypeStruct((1, NP), jnp.float32),
    )(parts, xt, *consts)


def kernel(x, edge_index, edge_weight, ggc_weight, gru_w_ih, gru_w_hh,
           gru_b_ih, gru_b_hh, lstm_w_ih, lstm_w_hh, lstm_b_ih, lstm_b_hh,
           lin_w, lin_b):
    n, f = x.shape
    e = edge_weight.shape[0]

    xt = jnp.zeros((4, NP), jnp.float32).at[:, :n].set(x.T)

    nr = -(-e // L)
    tr = -(-(-(-nr // NW)) // CK) * CK
    if (tr // CK) % 2 == 0:
        tr += CK
    nr2 = NW * tr
    pad_e = nr2 * L - e
    pad_i = lax.iota(jnp.int32, pad_e)
    src = jnp.concatenate([edge_index[0], pad_i % n])
    dst = jnp.concatenate([edge_index[1], n + pad_i % (NP - n)])
    w = jnp.concatenate([edge_weight, jnp.zeros((pad_e,), jnp.float32)])
    zeros = jnp.zeros((ZR,), jnp.float32)

    xb = lax.bitcast_convert_type(x.astype(jnp.bfloat16),
                                  jnp.uint16).astype(jnp.uint32)
    xq = jnp.zeros((2, NP), jnp.uint32)
    xq = xq.at[0, :n].set((xb[:, 0] << 16) | xb[:, 1])
    xq = xq.at[1, :n].set((xb[:, 2] << 16) | xb[:, 3])
    xq = lax.bitcast_convert_type(xq, jnp.int32)
    parts = _make_sc_agg(tr)(xq, src, dst, w, zeros)
    parts = parts.reshape(2, 5, NP)

    col = lambda v: v.reshape(-1, 1)
    consts = [
        ggc_weight.T,
        gru_w_ih[0:4], gru_w_hh[0:4], col(gru_b_ih[0:4] + gru_b_hh[0:4]),
        gru_w_ih[4:8], gru_w_hh[4:8], col(gru_b_ih[4:8] + gru_b_hh[4:8]),
        gru_w_ih[8:12], col(gru_b_ih[8:12]),
        gru_w_hh[8:12], col(gru_b_hh[8:12]),
        lstm_w_ih[0:32], col(lstm_b_ih[0:32] + lstm_b_hh[0:32]),
        lstm_w_ih[64:96], col(lstm_b_ih[64:96] + lstm_b_hh[64:96]),
        lstm_w_ih[96:128], col(lstm_b_ih[96:128] + lstm_b_hh[96:128]),
        lin_w, lin_b.reshape(1, 1),
    ]
    out_t = _dense_call(parts, xt, consts, bn=2048)
    return out_t.reshape(NP, 1)[:n]


# speedup vs baseline: 1.2965x; 1.1240x over previous
import jax
import jax.numpy as jnp
from jax import lax
from jax.experimental import pallas as pl
from jax.experimental.pallas import tpu as pltpu
from jax.experimental.pallas import tpu_sc as plsc

import functools

NP = 102400
L = 128
NW = 32
CK = 32
ZR = NP // 16


def _sc_agg_body(xq_h, ei_h, w_h, as_h, ad_h, aw_h, zeros_h, out,
                 xq0, xq1, a0, a1, a2, a3, a4,
                 s0, s1, d0, d1, w0, w1,
                 q00, q01, q10, q11,
                 c00, c01, c02, c03, c10, c11, c12, c13, ones_v,
                 gsem0, gsem1, ssem0, ssem1, esem0, esem1,
                 tr, nchunk, ne):
    c = lax.axis_index("c")
    s = lax.axis_index("s")
    xs = [xq0, xq1]
    ac = [a0, a1, a2, a3, a4]
    srcb = [s0, s1]
    dstb = [d0, d1]
    wb = [w0, w1]
    qcols = [[q00, q01], [q10, q11]]
    wid = s * 2 + c
    ebase0 = wid * tr * L
    cols = [[c00, c01, c02, c03], [c10, c11, c12, c13]]
    gsem = [gsem0, gsem1]
    ssem = [ssem0, ssem1]
    esem = [esem0, esem1]
    ce = CK * L

    for f in range(2):
        pltpu.sync_copy(xq_h.at[f, pl.ds(s * ZR, ZR)],
                        xs[f].at[pl.ds(s * ZR, ZR)])
    for f in range(5):
        pltpu.sync_copy(zeros_h, ac[f].at[pl.ds(s * ZR, ZR)])
    i16 = lax.broadcasted_iota(jnp.int32, (16,), 0)
    one16 = (i16 * 0 + 1).astype(jnp.float32)

    def fill(u, carry):
        ones_v[pl.ds(u * 16, 16)] = one16
        return carry

    lax.fori_loop(0, ce // 16, fill, 0)
    plsc.subcore_barrier()

    def fire_stage(ci, b):
        base = ebase0 + ci * ce
        abase = ci * ce

        @pl.when(wid < NW - 1)
        def _():
            pltpu.async_copy(ei_h.at[pl.ds(base, ce)], srcb[b], esem[b])
            pltpu.async_copy(ei_h.at[pl.ds(ne + base, ce)], dstb[b],
                             esem[b])
            pltpu.async_copy(w_h.at[pl.ds(base, ce)], wb[b], esem[b])

        @pl.when(wid == NW - 1)
        def _():
            pltpu.async_copy(as_h.at[pl.ds(abase, ce)], srcb[b], esem[b])
            pltpu.async_copy(ad_h.at[pl.ds(abase, ce)], dstb[b], esem[b])
            pltpu.async_copy(aw_h.at[pl.ds(abase, ce)], wb[b], esem[b])

    def wait_stage(b):
        pltpu.make_async_copy(ei_h.at[pl.ds(0, ce)], srcb[b], esem[b]).wait()
        pltpu.make_async_copy(ei_h.at[pl.ds(0, ce)], dstb[b], esem[b]).wait()
        pltpu.make_async_copy(w_h.at[pl.ds(0, ce)], wb[b], esem[b]).wait()

    def fire_gathers(b):
        for f in range(2):
            pltpu.async_copy(xs[f].at[srcb[b]], qcols[b][f], gsem[b])

    def wait_gathers(b):
        for f in range(2):
            pltpu.make_async_copy(xs[f].at[srcb[b]], qcols[b][f],
                                  gsem[b]).wait()

    def fire_scatters(b):
        for f in range(4):
            pltpu.async_copy(cols[b][f], ac[f].at[dstb[b]], ssem[b],
                             add=True)
        pltpu.async_copy(ones_v, ac[4].at[dstb[b]], ssem[b], add=True)

    def wait_scatters(b):
        for f in range(4):
            pltpu.make_async_copy(cols[b][f], ac[f].at[dstb[b]],
                                  ssem[b]).wait()
        pltpu.make_async_copy(ones_v, ac[4].at[dstb[b]], ssem[b]).wait()

    i16v = lax.broadcasted_iota(jnp.int32, (16,), 0)
    himask = i16v * 0 + (-65536)
    sh16 = i16v * 0 + 16

    def multiply(b):
        def mul_body(j, carry2):
            for u in range(L // 16):
                o = j * L + u * 16
                wv = wb[b][pl.ds(o, 16)]
                for p in range(2):
                    q = qcols[b][p][pl.ds(o, 16)]
                    hi = plsc.bitcast(q & himask, jnp.float32)
                    lo = plsc.bitcast(q << sh16, jnp.float32)
                    cols[b][2 * p][pl.ds(o, 16)] = hi * wv
                    cols[b][2 * p + 1][pl.ds(o, 16)] = lo * wv
            return carry2

        lax.fori_loop(0, CK, mul_body, 0)

    def step(ci, b, first):
        b2 = 1 - b
        if not first:
            wait_scatters(b2)

        @pl.when(ci + 1 < nchunk)
        def _():
            fire_stage(ci + 1, b2)

        wait_gathers(b)
        multiply(b)
        fire_scatters(b)

        @pl.when(ci + 1 < nchunk)
        def _():
            wait_stage(b2)
            fire_gathers(b2)

    fire_stage(0, 0)
    wait_stage(0)
    fire_gathers(0)
    step(0, 0, True)

    def pair_body(k, carry):
        step(2 * k + 1, 1, False)
        step(2 * k + 2, 0, False)
        return carry

    lax.fori_loop(0, (nchunk - 1) // 2, pair_body, 0)
    wait_scatters(0)

    plsc.subcore_barrier()
    for f in range(5):
        pltpu.sync_copy(ac[f].at[pl.ds(s * ZR, ZR)],
                        out.at[c * 5 + f, pl.ds(s * ZR, ZR)])


def _make_sc_agg(tr, ne):
    mesh = plsc.VectorSubcoreMesh(core_axis_name="c", subcore_axis_name="s",
                                  num_cores=2, num_subcores=16)
    body = functools.partial(_sc_agg_body, tr=tr, nchunk=tr // CK, ne=ne)
    return pl.kernel(
        body,
        out_type=jax.ShapeDtypeStruct((10, NP), jnp.float32),
        mesh=mesh,
        compiler_params=pltpu.CompilerParams(needs_layout_passes=False),
        scratch_types=(
            [pltpu.VMEM_SHARED((NP,), jnp.int32) for _ in range(2)]
            + [pltpu.VMEM_SHARED((NP,), jnp.float32) for _ in range(5)]
            + [pltpu.VMEM((CK * L,), jnp.int32) for _ in range(2)]
            + [pltpu.VMEM((CK * L,), jnp.int32) for _ in range(2)]
            + [pltpu.VMEM((CK * L,), jnp.float32) for _ in range(2)]
            + [pltpu.VMEM((CK * L,), jnp.int32) for _ in range(4)]
            + [pltpu.VMEM((CK * L,), jnp.float32) for _ in range(8)]
            + [pltpu.VMEM((CK * L,), jnp.float32)]
            + [pltpu.SemaphoreType.DMA for _ in range(6)]),
        name="gcn_edge_aggregate",
    )


def _dense_body(parts_ref, xt_ref, wg_ref,
                wri_ref, wrh_ref, br_ref, wzi_ref, wzh_ref, bz_ref,
                wni_ref, bni_ref, wnh_ref, bnh_ref,
                wi_ref, bi_ref, wg2_ref, bg2_ref, wo_ref, bo_ref,
                lw_ref, lb_ref, out_ref):
    seg = parts_ref[0] + parts_ref[1]
    cnt = jnp.clip(seg[4:5], 1.0, None)

    def mm(w_ref, v):
        return lax.dot_general(w_ref[...], v, (((1,), (0,)), ((), ())),
                               preferred_element_type=jnp.float32)

    agg = mm(wg_ref, seg[0:4] / cnt)
    xt = xt_ref[...]
    r = jax.nn.sigmoid(mm(wri_ref, agg) + mm(wrh_ref, xt) + br_ref[...])
    z = jax.nn.sigmoid(mm(wzi_ref, agg) + mm(wzh_ref, xt) + bz_ref[...])
    n = jnp.tanh(mm(wni_ref, agg) + bni_ref[...]
                 + r * (mm(wnh_ref, xt) + bnh_ref[...]))
    h = (1.0 - z) * n + z * xt
    ig = jax.nn.sigmoid(mm(wi_ref, h) + bi_ref[...])
    gg = jnp.tanh(mm(wg2_ref, h) + bg2_ref[...])
    og = jax.nn.sigmoid(mm(wo_ref, h) + bo_ref[...])
    hout = og * jnp.tanh(ig * gg)
    out_ref[...] = (lax.dot_general(lw_ref[...], jnp.maximum(hout, 0.0),
                                    (((1,), (0,)), ((), ())),
                                    preferred_element_type=jnp.float32)
                    + lb_ref[...])


def _dense_call(parts, xt, consts, bn):
    grid = NP // bn
    small = [pl.BlockSpec(c.shape, lambda i, nd=c.ndim: (0,) * nd)
             for c in consts]
    return pl.pallas_call(
        _dense_body,
        grid=(grid,),
        in_specs=[
            pl.BlockSpec((2, 5, bn), lambda i: (0, 0, i)),
            pl.BlockSpec((4, bn), lambda i: (0, i)),
        ] + small,
        out_specs=pl.BlockSpec((1, bn), lambda i: (0, i)),
        out_shape=jax.ShapeDtypeStruct((1, NP), jnp.float32),
    )(parts, xt, *consts)


def kernel(x, edge_index, edge_weight, ggc_weight, gru_w_ih, gru_w_hh,
           gru_b_ih, gru_b_hh, lstm_w_ih, lstm_w_hh, lstm_b_ih, lstm_b_hh,
           lin_w, lin_b):
    n, f = x.shape
    e = edge_weight.shape[0]

    xt = jnp.zeros((4, NP), jnp.float32).at[:, :n].set(x.T)

    nr = -(-e // L)
    tr = -(-(-(-nr // NW)) // CK) * CK
    if (tr // CK) % 2 == 0:
        tr += CK
    nr2 = NW * tr
    pad_e = nr2 * L - e
    te = tr * L
    pad_i = lax.iota(jnp.int32, pad_e)
    a0 = te * (NW - 1)
    aux_src = jnp.concatenate([edge_index[0, a0:], pad_i % n])
    aux_dst = jnp.concatenate([edge_index[1, a0:], n + pad_i % (NP - n)])
    aux_w = jnp.concatenate([edge_weight[a0:],
                             jnp.zeros((pad_e,), jnp.float32)])
    ei_flat = edge_index.reshape(2 * e)
    zeros = jnp.zeros((ZR,), jnp.float32)

    xb = lax.bitcast_convert_type(x.astype(jnp.bfloat16),
                                  jnp.uint16).astype(jnp.uint32)
    xq = jnp.zeros((2, NP), jnp.uint32)
    xq = xq.at[0, :n].set((xb[:, 0] << 16) | xb[:, 1])
    xq = xq.at[1, :n].set((xb[:, 2] << 16) | xb[:, 3])
    xq = lax.bitcast_convert_type(xq, jnp.int32)
    parts = _make_sc_agg(tr, e)(xq, ei_flat, edge_weight, aux_src,
                                aux_dst, aux_w, zeros)
    parts = parts.reshape(2, 5, NP)

    col = lambda v: v.reshape(-1, 1)
    consts = [
        ggc_weight.T,
        gru_w_ih[0:4], gru_w_hh[0:4], col(gru_b_ih[0:4] + gru_b_hh[0:4]),
        gru_w_ih[4:8], gru_w_hh[4:8], col(gru_b_ih[4:8] + gru_b_hh[4:8]),
        gru_w_ih[8:12], col(gru_b_ih[8:12]),
        gru_w_hh[8:12], col(gru_b_hh[8:12]),
        lstm_w_ih[0:32], col(lstm_b_ih[0:32] + lstm_b_hh[0:32]),
        lstm_w_ih[64:96], col(lstm_b_ih[64:96] + lstm_b_hh[64:96]),
        lstm_w_ih[96:128], col(lstm_b_ih[96:128] + lstm_b_hh[96:128]),
        lin_w, lin_b.reshape(1, 1),
    ]
    out_t = _dense_call(parts, xt, consts, bn=2048)
    return out_t.reshape(NP, 1)[:n]
